# baseline (speedup 1.0000x reference)
import functools

import jax
import jax.numpy as jnp
from jax import lax
from jax.experimental import pallas as pl
from jax.experimental.pallas import tpu as pltpu
from jax.experimental.pallas import tpu_sc as plsc


def _gelu_exact(x):
    return x * 0.5 * (1.0 + lax.erf(x * 0.7071067811865476))


def _silu(x):
    return x * jax.nn.sigmoid(x)


def _router_body(x_ref, wr1_ref, br1_ref, g_ref, b_ref, wr2_ref, br2_ref,
                 ti_ref, tv_ref):
    x = x_ref[...]
    h = jnp.dot(x, wr1_ref[...], preferred_element_type=jnp.float32)
    h = h + br1_ref[...]
    mu = jnp.mean(h, axis=-1, keepdims=True)
    var = jnp.mean((h - mu) ** 2, axis=-1, keepdims=True)
    hn = (h - mu) * lax.rsqrt(var + 1e-5) * g_ref[...] + b_ref[...]
    hg = _gelu_exact(hn)
    logits = jnp.dot(hg, wr2_ref[...], preferred_element_type=jnp.float32)
    logits = logits + br2_ref[...]
    m = jnp.max(logits, axis=-1, keepdims=True)
    ex = jnp.exp(logits - m)
    sm = ex / jnp.sum(ex, axis=-1, keepdims=True)
    lanes = jax.lax.broadcasted_iota(jnp.int32, sm.shape, 1)
    big = jnp.int32(1 << 20)
    v1 = jnp.max(sm, axis=-1, keepdims=True)
    i1 = jnp.min(jnp.where(sm == v1, lanes, big), axis=-1, keepdims=True)
    sm2 = jnp.where(lanes == i1, -1.0, sm)
    v2 = jnp.max(sm2, axis=-1, keepdims=True)
    i2 = jnp.min(jnp.where(sm2 == v2, lanes, big), axis=-1, keepdims=True)
    ti_ref[...] = jnp.concatenate([i1, i2], axis=1)
    tv_ref[...] = jnp.concatenate([v1, v2], axis=1)


def _gmm_body(be_ref, xs_ref, w1_ref, b1_ref, w2_ref, b2_ref, out_ref,
              act_ref):
    e = be_ref[pl.program_id(0)]
    x = xs_ref[...].astype(jnp.bfloat16)
    h1 = jnp.dot(x, w1_ref[0], preferred_element_type=jnp.float32)
    h1 = h1 + b1_ref[0]

    @pl.when(e % 2 == 0)
    def _():
        act_ref[...] = _gelu_exact(h1).astype(jnp.bfloat16)

    @pl.when(e % 2 != 0)
    def _():
        act_ref[...] = _silu(h1).astype(jnp.bfloat16)

    y = jnp.dot(act_ref[...], w2_ref[0], preferred_element_type=jnp.float32)
    out_ref[...] = y + b2_ref[0]


def kernel(x, W1, b1, W2, b2, Wr1, br1, ln_g, ln_b, Wr2, br2, temp, bias):
    Bx, Lx, D = x.shape
    N = Bx * Lx
    E, _, DFF = W1.shape
    D2 = Wr1.shape[1]
    xf = x.reshape(N, D)

    wr2 = Wr2 / temp[0]
    br2f = (br2 / temp[0] + bias).reshape(1, E)

    BR = 2048
    top_i, top_v = pl.pallas_call(
        _router_body,
        grid=(N // BR,),
        in_specs=[
            pl.BlockSpec((BR, D), lambda i: (i, 0)),
            pl.BlockSpec((D, D2), lambda i: (0, 0)),
            pl.BlockSpec((1, D2), lambda i: (0, 0)),
            pl.BlockSpec((1, D2), lambda i: (0, 0)),
            pl.BlockSpec((1, D2), lambda i: (0, 0)),
            pl.BlockSpec((D2, E), lambda i: (0, 0)),
            pl.BlockSpec((1, E), lambda i: (0, 0)),
        ],
        out_specs=[
            pl.BlockSpec((BR, 2), lambda i: (i, 0)),
            pl.BlockSpec((BR, 2), lambda i: (i, 0)),
        ],
        out_shape=[
            jax.ShapeDtypeStruct((N, 2), jnp.int32),
            jax.ShapeDtypeStruct((N, 2), jnp.float32),
        ],
    )(xf, Wr1, br1.reshape(1, D2), ln_g.reshape(1, D2),
      ln_b.reshape(1, D2), wr2, br2f)

    BM = 256
    A = 2 * N
    PAD_N = (A + E * BM + 4095) // 4096 * 4096
    NB = PAD_N // BM

    ef = top_i.reshape(A)
    oh = (ef[:, None] == jnp.arange(E, dtype=jnp.int32)[None, :])
    cs = jnp.cumsum(oh.astype(jnp.int32), axis=0)
    rank = jnp.take_along_axis(cs, ef[:, None], axis=1)[:, 0] - 1
    counts = cs[-1]
    padded = (counts + BM - 1) // BM * BM
    off = jnp.concatenate(
        [jnp.zeros((1,), jnp.int32), jnp.cumsum(padded)[:-1].astype(jnp.int32)])
    pos = off[ef] + rank
    blk_e = jnp.clip(
        jnp.searchsorted(off, jnp.arange(NB, dtype=jnp.int32) * BM,
                         side="right").astype(jnp.int32) - 1, 0, E - 1)
    p1 = pos[0::2]
    p2 = pos[1::2]

    info = plsc.get_sparse_core_info()
    NC, NS = info.num_cores, info.num_subcores
    NW = NC * NS
    mesh = plsc.VectorSubcoreMesh(core_axis_name="c", subcore_axis_name="s")
    C = 128

    CG = 64

    def _sc_scatter_body(src_hbm, p1_hbm, p2_hbm, out_hbm, i1_v, i2_v,
                         buf0, buf1, rsem0, rsem1, s1sem, s2sem):
        wid = lax.axis_index("s") * NC + lax.axis_index("c")
        rows_pw = N // NW
        base_w = wid * rows_pw
        n = rows_pw // (2 * CG)
        for c in range(n):
            base = base_w + 2 * c * CG
            ra = pltpu.async_copy(src_hbm.at[pl.ds(base, CG)], buf0, rsem0)
            rb = pltpu.async_copy(src_hbm.at[pl.ds(base + CG, CG)], buf1,
                                  rsem1)
            pltpu.sync_copy(p1_hbm.at[pl.ds(base, CG)], i1_v)
            pltpu.sync_copy(p2_hbm.at[pl.ds(base, CG)], i2_v)
            ra.wait()
            s1 = pltpu.async_copy(buf0, out_hbm.at[i1_v], s1sem)
            s2 = pltpu.async_copy(buf0, out_hbm.at[i2_v], s2sem)
            s1.wait()
            s2.wait()
            pltpu.sync_copy(p1_hbm.at[pl.ds(base + CG, CG)], i1_v)
            pltpu.sync_copy(p2_hbm.at[pl.ds(base + CG, CG)], i2_v)
            rb.wait()
            s1 = pltpu.async_copy(buf1, out_hbm.at[i1_v], s1sem)
            s2 = pltpu.async_copy(buf1, out_hbm.at[i2_v], s2sem)
            s1.wait()
            s2.wait()

    xs = pl.kernel(
        _sc_scatter_body,
        out_type=jax.ShapeDtypeStruct((PAD_N, D), jnp.float32),
        mesh=mesh,
        scratch_types=[
            pltpu.VMEM((CG,), jnp.int32),
            pltpu.VMEM((CG,), jnp.int32),
            pltpu.VMEM((CG, D), jnp.float32),
            pltpu.VMEM((CG, D), jnp.float32),
            pltpu.SemaphoreType.DMA,
            pltpu.SemaphoreType.DMA,
            pltpu.SemaphoreType.DMA,
            pltpu.SemaphoreType.DMA,
        ],
    )(xf, p1, p2)

    ys = pl.pallas_call(
        _gmm_body,
        grid_spec=pltpu.PrefetchScalarGridSpec(
            num_scalar_prefetch=1,
            grid=(NB,),
            in_specs=[
                pl.BlockSpec((BM, D), lambda b, be: (b, 0)),
                pl.BlockSpec((1, D, DFF), lambda b, be: (be[b], 0, 0)),
                pl.BlockSpec((1, 1, DFF), lambda b, be: (be[b], 0, 0)),
                pl.BlockSpec((1, DFF, D), lambda b, be: (be[b], 0, 0)),
                pl.BlockSpec((1, 1, D), lambda b, be: (be[b], 0, 0)),
            ],
            out_specs=pl.BlockSpec((BM, D), lambda b, be: (b, 0)),
            scratch_shapes=[pltpu.VMEM((BM, DFF), jnp.bfloat16)],
        ),
        out_shape=jax.ShapeDtypeStruct((PAD_N, D), jnp.float32),
        compiler_params=pltpu.CompilerParams(
            dimension_semantics=("arbitrary",),
        ),
    )(blk_e, xs, W1.astype(jnp.bfloat16), b1.reshape(E, 1, DFF),
      W2.astype(jnp.bfloat16), b2.reshape(E, 1, D))

    CC = 64
    v1 = jnp.broadcast_to(top_v[:, 0:1], (N, 16))
    v2 = jnp.broadcast_to(top_v[:, 1:2], (N, 16))

    def _sc_combine_body(ys_hbm, i1_hbm, i2_hbm, v1_hbm, v2_hbm, out_hbm,
                         i1_v, i2_v, v1_v, v2_v, buf1, buf2, sem1, sem2):
        wid = lax.axis_index("s") * NC + lax.axis_index("c")
        rows_pw = N // NW
        base_w = wid * rows_pw
        for c in range(rows_pw // CC):
            base = base_w + c * CC
            pltpu.sync_copy(i1_hbm.at[pl.ds(base, CC)], i1_v)
            pltpu.sync_copy(i2_hbm.at[pl.ds(base, CC)], i2_v)
            cp1 = pltpu.async_copy(ys_hbm.at[i1_v], buf1, sem1)
            cp2 = pltpu.async_copy(ys_hbm.at[i2_v], buf2, sem2)
            pltpu.sync_copy(v1_hbm.at[pl.ds(base, CC)], v1_v)
            pltpu.sync_copy(v2_hbm.at[pl.ds(base, CC)], v2_v)
            cp1.wait()
            cp2.wait()

            @plsc.parallel_loop(0, CC, step=1, unroll=2)
            def _row(r):
                w1r = v1_v[r, :]
                w2r = v2_v[r, :]
                for dd in range(D // 16):
                    sl = pl.ds(dd * 16, 16)
                    buf1[r, sl] = w1r * buf1[r, sl] + w2r * buf2[r, sl]
            pltpu.sync_copy(buf1, out_hbm.at[pl.ds(base, CC)])

    out = pl.kernel(
        _sc_combine_body,
        out_type=jax.ShapeDtypeStruct((N, D), jnp.float32),
        mesh=mesh,
        scratch_types=[
            pltpu.VMEM((CC,), jnp.int32),
            pltpu.VMEM((CC,), jnp.int32),
            pltpu.VMEM((CC, 16), jnp.float32),
            pltpu.VMEM((CC, 16), jnp.float32),
            pltpu.VMEM((CC, D), jnp.float32),
            pltpu.VMEM((CC, D), jnp.float32),
            pltpu.SemaphoreType.DMA,
            pltpu.SemaphoreType.DMA,
        ],
    )(ys, p1, p2, v1, v2)

    return out.reshape(Bx, Lx, D)


# speedup vs baseline: 1.1968x; 1.1968x over previous
import functools

import jax
import jax.numpy as jnp
from jax import lax
from jax.experimental import pallas as pl
from jax.experimental.pallas import tpu as pltpu
from jax.experimental.pallas import tpu_sc as plsc


def _gelu_exact(x):
    return x * 0.5 * (1.0 + lax.erf(x * 0.7071067811865476))


def _silu(x):
    return x * jax.nn.sigmoid(x)


def _router_body(x_ref, wr1_ref, br1_ref, g_ref, b_ref, wr2_ref, br2_ref,
                 ti_ref, tv_ref):
    x = x_ref[...]
    h = jnp.dot(x, wr1_ref[...], preferred_element_type=jnp.float32)
    h = h + br1_ref[...]
    mu = jnp.mean(h, axis=-1, keepdims=True)
    var = jnp.mean((h - mu) ** 2, axis=-1, keepdims=True)
    hn = (h - mu) * lax.rsqrt(var + 1e-5) * g_ref[...] + b_ref[...]
    hg = _gelu_exact(hn)
    logits = jnp.dot(hg, wr2_ref[...], preferred_element_type=jnp.float32)
    logits = logits + br2_ref[...]
    m = jnp.max(logits, axis=-1, keepdims=True)
    ex = jnp.exp(logits - m)
    sm = ex / jnp.sum(ex, axis=-1, keepdims=True)
    lanes = jax.lax.broadcasted_iota(jnp.int32, sm.shape, 1)
    big = jnp.int32(1 << 20)
    v1 = jnp.max(sm, axis=-1, keepdims=True)
    i1 = jnp.min(jnp.where(sm == v1, lanes, big), axis=-1, keepdims=True)
    sm2 = jnp.where(lanes == i1, -1.0, sm)
    v2 = jnp.max(sm2, axis=-1, keepdims=True)
    i2 = jnp.min(jnp.where(sm2 == v2, lanes, big), axis=-1, keepdims=True)
    ti_ref[...] = jnp.concatenate([i1, i2], axis=1)
    tv_ref[...] = jnp.concatenate([v1, v2], axis=1)


def _gmm_body(be_ref, xs_ref, w1_ref, b1_ref, w2_ref, b2_ref, out_ref):
    e = be_ref[pl.program_id(0)]
    x = xs_ref[...].astype(jnp.bfloat16)
    h1 = jnp.dot(x, w1_ref[0], preferred_element_type=jnp.float32)
    h1 = h1 + b1_ref[0]
    a = jnp.where(e % 2 == 0, _gelu_exact(h1), _silu(h1))
    y = jnp.dot(a.astype(jnp.bfloat16), w2_ref[0],
                preferred_element_type=jnp.float32)
    out_ref[...] = y + b2_ref[0]


def kernel(x, W1, b1, W2, b2, Wr1, br1, ln_g, ln_b, Wr2, br2, temp, bias):
    Bx, Lx, D = x.shape
    N = Bx * Lx
    E, _, DFF = W1.shape
    D2 = Wr1.shape[1]
    xf = x.reshape(N, D)

    wr2 = Wr2 / temp[0]
    br2f = (br2 / temp[0] + bias).reshape(1, E)

    BR = 2048
    top_i, top_v = pl.pallas_call(
        _router_body,
        grid=(N // BR,),
        in_specs=[
            pl.BlockSpec((BR, D), lambda i: (i, 0)),
            pl.BlockSpec((D, D2), lambda i: (0, 0)),
            pl.BlockSpec((1, D2), lambda i: (0, 0)),
            pl.BlockSpec((1, D2), lambda i: (0, 0)),
            pl.BlockSpec((1, D2), lambda i: (0, 0)),
            pl.BlockSpec((D2, E), lambda i: (0, 0)),
            pl.BlockSpec((1, E), lambda i: (0, 0)),
        ],
        out_specs=[
            pl.BlockSpec((BR, 2), lambda i: (i, 0)),
            pl.BlockSpec((BR, 2), lambda i: (i, 0)),
        ],
        out_shape=[
            jax.ShapeDtypeStruct((N, 2), jnp.int32),
            jax.ShapeDtypeStruct((N, 2), jnp.float32),
        ],
    )(xf, Wr1, br1.reshape(1, D2), ln_g.reshape(1, D2),
      ln_b.reshape(1, D2), wr2, br2f)

    BM = 512
    A = 2 * N
    PAD_N = (A + E * BM + 4095) // 4096 * 4096
    NB = PAD_N // BM

    ef = top_i.reshape(A)
    oh = (ef[:, None] == jnp.arange(E, dtype=jnp.int32)[None, :])
    cs = jnp.cumsum(oh.astype(jnp.int32), axis=0)
    rank = jnp.take_along_axis(cs, ef[:, None], axis=1)[:, 0] - 1
    counts = cs[-1]
    padded = (counts + BM - 1) // BM * BM
    off = jnp.concatenate(
        [jnp.zeros((1,), jnp.int32), jnp.cumsum(padded)[:-1].astype(jnp.int32)])
    pos = off[ef] + rank
    blk_e = jnp.clip(
        jnp.searchsorted(off, jnp.arange(NB, dtype=jnp.int32) * BM,
                         side="right").astype(jnp.int32) - 1, 0, E - 1)
    p1 = pos[0::2]
    p2 = pos[1::2]

    info = plsc.get_sparse_core_info()
    NC, NS = info.num_cores, info.num_subcores
    NW = NC * NS
    mesh = plsc.VectorSubcoreMesh(core_axis_name="c", subcore_axis_name="s")
    C = 128

    CG = 64

    def _sc_scatter_body(src_hbm, p1_hbm, p2_hbm, out_hbm, i1_v, i2_v,
                         buf0, buf1, rsem0, rsem1, s1sem, s2sem):
        wid = lax.axis_index("s") * NC + lax.axis_index("c")
        rows_pw = N // NW
        base_w = wid * rows_pw
        n = rows_pw // (2 * CG)
        for c in range(n):
            base = base_w + 2 * c * CG
            ra = pltpu.async_copy(src_hbm.at[pl.ds(base, CG)], buf0, rsem0)
            rb = pltpu.async_copy(src_hbm.at[pl.ds(base + CG, CG)], buf1,
                                  rsem1)
            pltpu.sync_copy(p1_hbm.at[pl.ds(base, CG)], i1_v)
            pltpu.sync_copy(p2_hbm.at[pl.ds(base, CG)], i2_v)
            ra.wait()
            s1 = pltpu.async_copy(buf0, out_hbm.at[i1_v], s1sem)
            s2 = pltpu.async_copy(buf0, out_hbm.at[i2_v], s2sem)
            s1.wait()
            s2.wait()
            pltpu.sync_copy(p1_hbm.at[pl.ds(base + CG, CG)], i1_v)
            pltpu.sync_copy(p2_hbm.at[pl.ds(base + CG, CG)], i2_v)
            rb.wait()
            s1 = pltpu.async_copy(buf1, out_hbm.at[i1_v], s1sem)
            s2 = pltpu.async_copy(buf1, out_hbm.at[i2_v], s2sem)
            s1.wait()
            s2.wait()

    xs = pl.kernel(
        _sc_scatter_body,
        out_type=jax.ShapeDtypeStruct((PAD_N, D), jnp.float32),
        mesh=mesh,
        scratch_types=[
            pltpu.VMEM((CG,), jnp.int32),
            pltpu.VMEM((CG,), jnp.int32),
            pltpu.VMEM((CG, D), jnp.float32),
            pltpu.VMEM((CG, D), jnp.float32),
            pltpu.SemaphoreType.DMA,
            pltpu.SemaphoreType.DMA,
            pltpu.SemaphoreType.DMA,
            pltpu.SemaphoreType.DMA,
        ],
    )(xf, p1, p2)

    ys = pl.pallas_call(
        _gmm_body,
        grid_spec=pltpu.PrefetchScalarGridSpec(
            num_scalar_prefetch=1,
            grid=(NB,),
            in_specs=[
                pl.BlockSpec((BM, D), lambda b, be: (b, 0)),
                pl.BlockSpec((1, D, DFF), lambda b, be: (be[b], 0, 0)),
                pl.BlockSpec((1, 1, DFF), lambda b, be: (be[b], 0, 0)),
                pl.BlockSpec((1, DFF, D), lambda b, be: (be[b], 0, 0)),
                pl.BlockSpec((1, 1, D), lambda b, be: (be[b], 0, 0)),
            ],
            out_specs=pl.BlockSpec((BM, D), lambda b, be: (b, 0)),
        ),
        out_shape=jax.ShapeDtypeStruct((PAD_N, D), jnp.float32),
        compiler_params=pltpu.CompilerParams(
            dimension_semantics=("arbitrary",),
        ),
    )(blk_e, xs, W1.astype(jnp.bfloat16), b1.reshape(E, 1, DFF),
      W2.astype(jnp.bfloat16), b2.reshape(E, 1, D))

    CC = 64
    v1 = jnp.broadcast_to(top_v[:, 0:1], (N, 16))
    v2 = jnp.broadcast_to(top_v[:, 1:2], (N, 16))

    def _sc_combine_body(ys_hbm, i1_hbm, i2_hbm, v1_hbm, v2_hbm, out_hbm,
                         i1_v, i2_v, v1_v, v2_v, buf1, buf2, sem1, sem2):
        wid = lax.axis_index("s") * NC + lax.axis_index("c")
        rows_pw = N // NW
        base_w = wid * rows_pw
        for c in range(rows_pw // CC):
            base = base_w + c * CC
            pltpu.sync_copy(i1_hbm.at[pl.ds(base, CC)], i1_v)
            pltpu.sync_copy(i2_hbm.at[pl.ds(base, CC)], i2_v)
            cp1 = pltpu.async_copy(ys_hbm.at[i1_v], buf1, sem1)
            cp2 = pltpu.async_copy(ys_hbm.at[i2_v], buf2, sem2)
            pltpu.sync_copy(v1_hbm.at[pl.ds(base, CC)], v1_v)
            pltpu.sync_copy(v2_hbm.at[pl.ds(base, CC)], v2_v)
            cp1.wait()
            cp2.wait()

            @plsc.parallel_loop(0, CC, step=1, unroll=2)
            def _row(r):
                w1r = v1_v[r, :]
                w2r = v2_v[r, :]
                for dd in range(D // 16):
                    sl = pl.ds(dd * 16, 16)
                    buf1[r, sl] = w1r * buf1[r, sl] + w2r * buf2[r, sl]
            pltpu.sync_copy(buf1, out_hbm.at[pl.ds(base, CC)])

    out = pl.kernel(
        _sc_combine_body,
        out_type=jax.ShapeDtypeStruct((N, D), jnp.float32),
        mesh=mesh,
        scratch_types=[
            pltpu.VMEM((CC,), jnp.int32),
            pltpu.VMEM((CC,), jnp.int32),
            pltpu.VMEM((CC, 16), jnp.float32),
            pltpu.VMEM((CC, 16), jnp.float32),
            pltpu.VMEM((CC, D), jnp.float32),
            pltpu.VMEM((CC, D), jnp.float32),
            pltpu.SemaphoreType.DMA,
            pltpu.SemaphoreType.DMA,
        ],
    )(ys, p1, p2, v1, v2)

    return out.reshape(Bx, Lx, D)
